# Initial kernel scaffold; baseline (speedup 1.0000x reference)
import jax
import jax.numpy as jnp
from jax.experimental import pallas as pl


def kernel(x, weight):
    raise NotImplementedError("write your pallas kernel here")


# speedup vs baseline: 3.3086x; 3.3086x over previous
import functools

import jax
import jax.numpy as jnp
import numpy as np
from jax import lax
from jax.experimental import pallas as pl
from jax.experimental.pallas import tpu as pltpu
from jax.experimental.pallas import tpu_sc as plsc

_FIELD_DIMS = [1000] * 26
_EMBED_DIM = 128
_BATCH = 16384
_NUM_FIELDS = 26
_OFF = np.array((0, *np.cumsum(_FIELD_DIMS)[:-1]), dtype=np.int32)

_NC = 2
_NS = 16
_NW = _NC * _NS
_TOTAL = _BATCH * _NUM_FIELDS
_ROWS_W = _TOTAL // _NW
_BLK = 128
_NBLK = _ROWS_W // _BLK
_XROWS = _TOTAL // _BLK
_XROWS_W = _XROWS // _NW
_PAT = 208
_ROWS_PER_PAT = _PAT // 16


def _body(x_hbm, off_hbm, w_hbm, out_hbm, x_v, idx_v, off_v, rows0, rows1,
          gsem):
    c = lax.axis_index("c")
    s = lax.axis_index("s")
    wid = s * _NC + c
    xrow0 = wid * _XROWS_W
    out0 = wid * _ROWS_W

    pltpu.sync_copy(x_hbm.at[pl.ds(xrow0, _XROWS_W)], x_v)
    pltpu.sync_copy(off_hbm, off_v)

    def compute_idx(j, carry):
        for t in range(_ROWS_PER_PAT):
            r = j * _ROWS_PER_PAT + t
            for c8 in range(8):
                st = (t * 128 + c8 * 16) % _PAT
                sl = pl.ds(c8 * 16, 16)
                idx_v[r, sl] = x_v[r, sl] + off_v[pl.ds(st, 16)]
        return carry

    lax.fori_loop(0, _XROWS_W // _ROWS_PER_PAT, compute_idx, 0)

    pltpu.async_copy(w_hbm.at[idx_v.at[0]], rows0, gsem)
    pltpu.async_copy(w_hbm.at[idx_v.at[1]], rows1, gsem)

    def step(i, carry):
        for b, rows_v in ((0, rows0), (1, rows1)):
            g = 2 * i + b
            pltpu.make_async_copy(w_hbm.at[idx_v.at[g]], rows_v, gsem).wait()
            pltpu.sync_copy(rows_v, out_hbm.at[pl.ds(out0 + g * _BLK, _BLK)])

            @pl.when(i < _NBLK // 2 - 1)
            def _():
                pltpu.async_copy(w_hbm.at[idx_v.at[g + 2]], rows_v, gsem)

        return carry

    lax.fori_loop(0, _NBLK // 2, step, 0)


@jax.jit
def kernel(x, weight):
    x2d = x.reshape(_XROWS, _BLK)
    off = jnp.tile(jnp.asarray(_OFF), _PAT // _NUM_FIELDS)
    mesh = plsc.VectorSubcoreMesh(core_axis_name="c", subcore_axis_name="s")
    out = pl.kernel(
        _body,
        out_type=jax.ShapeDtypeStruct((_TOTAL, _EMBED_DIM), jnp.float32),
        mesh=mesh,
        scratch_types=[
            pltpu.VMEM((_XROWS_W, _BLK), jnp.int32),
            pltpu.VMEM((_XROWS_W, _BLK), jnp.int32),
            pltpu.VMEM((_PAT,), jnp.int32),
            pltpu.VMEM((_BLK, _EMBED_DIM), jnp.float32),
            pltpu.VMEM((_BLK, _EMBED_DIM), jnp.float32),
            pltpu.SemaphoreType.DMA,
        ],
    )(x2d, off, weight)
    return out.reshape(_BATCH, _NUM_FIELDS, _EMBED_DIM)


# speedup vs baseline: 3.3324x; 1.0072x over previous
import functools

import jax
import jax.numpy as jnp
import numpy as np
from jax import lax
from jax.experimental import pallas as pl
from jax.experimental.pallas import tpu as pltpu
from jax.experimental.pallas import tpu_sc as plsc

_FIELD_DIMS = [1000] * 26
_EMBED_DIM = 128
_BATCH = 16384
_NUM_FIELDS = 26
_OFF = np.array((0, *np.cumsum(_FIELD_DIMS)[:-1]), dtype=np.int32)

_NC = 2
_NS = 16
_NW = _NC * _NS
_TOTAL = _BATCH * _NUM_FIELDS
_ROWS_W = _TOTAL // _NW
_BLK = 128
_NBLK = _ROWS_W // _BLK
_XROWS = _TOTAL // _BLK
_XROWS_W = _XROWS // _NW
_PAT = 208
_ROWS_PER_PAT = _PAT // 16


def _body(x_hbm, off_hbm, w_hbm, out_hbm, x_v, off_v, rows0, rows1, rows2,
          rows3, gsem, wsem):
    c = lax.axis_index("c")
    s = lax.axis_index("s")
    wid = s * _NC + c
    xrow0 = wid * _XROWS_W
    out0 = wid * _ROWS_W
    rows = (rows0, rows1, rows2, rows3)

    pltpu.sync_copy(x_hbm.at[pl.ds(xrow0, _XROWS_W)], x_v)
    pltpu.sync_copy(off_hbm, off_v)

    def compute_idx(j, carry):
        for t in range(_ROWS_PER_PAT):
            r = j * _ROWS_PER_PAT + t
            for c8 in range(8):
                st = (t * 128 + c8 * 16) % _PAT
                sl = pl.ds(c8 * 16, 16)
                x_v[r, sl] = x_v[r, sl] + off_v[pl.ds(st, 16)]
        return carry

    lax.fori_loop(0, _XROWS_W // _ROWS_PER_PAT, compute_idx, 0)

    def out_at(g):
        return out_hbm.at[pl.ds(out0 + g * _BLK, _BLK)]

    pltpu.async_copy(w_hbm.at[x_v.at[0]], rows0, gsem)
    pltpu.async_copy(w_hbm.at[x_v.at[1]], rows1, gsem)

    def step(i, carry):
        for j in range(4):
            g = 4 * i + j
            pltpu.make_async_copy(w_hbm.at[x_v.at[g]], rows[j], gsem).wait()
            pltpu.async_copy(rows[j], out_at(g), wsem)
            drain = pltpu.make_async_copy(rows[j], out_at(g), wsem)
            if j < 2:

                @pl.when(i >= 1)
                def _():
                    drain.wait()

                pltpu.async_copy(w_hbm.at[x_v.at[g + 2]], rows[j + 2], gsem)
            else:
                drain.wait()

                @pl.when(i < _NBLK // 4 - 1)
                def _():
                    pltpu.async_copy(
                        w_hbm.at[x_v.at[g + 2]], rows[(j + 2) % 4], gsem)

        return carry

    lax.fori_loop(0, _NBLK // 4, step, 0)

    pltpu.make_async_copy(rows2, out_at(_NBLK - 2), wsem).wait()
    pltpu.make_async_copy(rows3, out_at(_NBLK - 1), wsem).wait()


@jax.jit
def kernel(x, weight):
    x2d = x.reshape(_XROWS, _BLK)
    off = jnp.tile(jnp.asarray(_OFF), _PAT // _NUM_FIELDS)
    mesh = plsc.VectorSubcoreMesh(core_axis_name="c", subcore_axis_name="s")
    out = pl.kernel(
        _body,
        out_type=jax.ShapeDtypeStruct((_TOTAL, _EMBED_DIM), jnp.float32),
        mesh=mesh,
        scratch_types=[
            pltpu.VMEM((_XROWS_W, _BLK), jnp.int32),
            pltpu.VMEM((_PAT,), jnp.int32),
            pltpu.VMEM((_BLK, _EMBED_DIM), jnp.float32),
            pltpu.VMEM((_BLK, _EMBED_DIM), jnp.float32),
            pltpu.VMEM((_BLK, _EMBED_DIM), jnp.float32),
            pltpu.VMEM((_BLK, _EMBED_DIM), jnp.float32),
            pltpu.SemaphoreType.DMA,
            pltpu.SemaphoreType.DMA,
        ],
    )(x2d, off, weight)
    return out.reshape(_BATCH, _NUM_FIELDS, _EMBED_DIM)
